# Initial kernel scaffold; baseline (speedup 1.0000x reference)
import jax
import jax.numpy as jnp
from jax.experimental import pallas as pl


def kernel(x, _):
    raise NotImplementedError("write your pallas kernel here")


# speedup vs baseline: 257.4165x; 257.4165x over previous
import functools

import jax
import jax.numpy as jnp
from jax import lax
from jax.experimental import pallas as pl
from jax.experimental.pallas import tpu as pltpu
from jax.experimental.pallas import tpu_sc as plsc

_N = 16384 * 1024
_NC, _NS, _L = 2, 16, 16
_NW = _NC * _NS
_PER_W = _N // _NW
_CHUNK = 32768
_NCHUNK = _PER_W // _CHUNK


@functools.partial(
    pl.kernel,
    mesh=plsc.VectorSubcoreMesh(core_axis_name="c", subcore_axis_name="s"),
    out_type=jax.ShapeDtypeStruct((_NW * _L,), jnp.float32),
    scratch_types=[
        pltpu.VMEM((2, _CHUNK), jnp.float32),
        pltpu.VMEM((_L,), jnp.float32),
        pltpu.SemaphoreType.DMA,
        pltpu.SemaphoreType.DMA,
    ],
)
def _sc_absmax(x_hbm, out_hbm, buf, part, sem0, sem1):
    wid = lax.axis_index("s") * _NC + lax.axis_index("c")
    base = wid * _PER_W
    sems = (sem0, sem1)

    handles = [None, None]
    handles[0] = pltpu.async_copy(
        x_hbm.at[pl.ds(base, _CHUNK)], buf.at[0], sems[0])
    acc = jnp.zeros((_L,), jnp.float32)
    for g in range(_NCHUNK):
        b = g % 2
        if g + 1 < _NCHUNK:
            handles[1 - b] = pltpu.async_copy(
                x_hbm.at[pl.ds(base + (g + 1) * _CHUNK, _CHUNK)],
                buf.at[1 - b], sems[1 - b])
        handles[b].wait()

        def body(i, a):
            v = buf[b, pl.ds(i * _L, _L)]
            return jnp.maximum(a, jnp.abs(v))

        acc = lax.fori_loop(0, _CHUNK // _L, body, acc, unroll=8)

    part[...] = acc
    pltpu.sync_copy(part, out_hbm.at[pl.ds(wid * _L, _L)])


def _finalize_body(p_ref, o_ref):
    m = jnp.max(p_ref[...])
    r = 1.0 / m
    bits = lax.bitcast_convert_type(r, jnp.int32)
    e = ((bits >> 23) & 0xFF) - 127
    o_ref[0, 0] = -jnp.clip(e.astype(jnp.float32), -15.0, 15.0)


def kernel(x, _):
    parts = _sc_absmax(x.reshape(_N))
    out = pl.pallas_call(
        _finalize_body,
        out_shape=jax.ShapeDtypeStruct((1, 1), jnp.float32),
        out_specs=pl.BlockSpec(memory_space=pltpu.SMEM),
    )(parts.reshape(_NW, _L))
    return out[0, 0]
